# baseline (speedup 1.0000x reference)
import functools

import jax
import jax.numpy as jnp
from jax import lax
from jax.experimental import pallas as pl
from jax.experimental.pallas import tpu as pltpu
from jax.experimental.pallas import tpu_sc as plsc

VOC = 100000
D = 64
B = 1024
CTX = 20

NC = 2
NS = 16
NW = NC * NS
L = 16
BPW = B // NW
IPW = BPW * CTX
ICHUNK = 128
NCHUNK = IPW // ICHUNK

_sc_mesh = plsc.VectorSubcoreMesh(core_axis_name="c", subcore_axis_name="s")


@functools.partial(
    pl.kernel,
    mesh=_sc_mesh,
    out_type=jax.ShapeDtypeStruct((B, D), jnp.float32),
    scratch_types=[
        pltpu.VMEM((NCHUNK, ICHUNK), jnp.int32),
        pltpu.VMEM((IPW, D), jnp.float32),
        pltpu.VMEM((BPW, D), jnp.float32),
        pltpu.SemaphoreType.DMA,
    ],
    compiler_params=pltpu.CompilerParams(use_tc_tiling_on_sc=False),
)
def _pool_sc(x_hbm, w_hbm, out_hbm, idx_v, rows_v, pool_v, sem):
    wid = lax.axis_index("s") * NC + lax.axis_index("c")
    pltpu.sync_copy(x_hbm.at[wid], idx_v)
    cps = [
        pltpu.async_copy(
            w_hbm.at[idx_v.at[j]],
            rows_v.at[pl.ds(j * ICHUNK, ICHUNK)],
            sem,
        )
        for j in range(NCHUNK)
    ]
    for cp in cps:
        cp.wait()

    inv_ctx = jnp.float32(1.0 / CTX)

    def body(b, carry):
        base = b * CTX
        for d in range(D // L):
            sl = pl.ds(d * L, L)
            acc = rows_v[base, sl]
            for j in range(1, CTX):
                acc = acc + rows_v[base + j, sl]
            pool_v[b, sl] = acc * inv_ctx
        return carry

    lax.fori_loop(0, BPW, body, 0)
    pltpu.sync_copy(pool_v, out_hbm.at[pl.ds(wid * BPW, BPW)])


NVBLK = 4096
_GRID = (VOC + NVBLK - 1) // NVBLK
_VOCPAD = ((VOC + 127) // 128) * 128
_LASTW = VOC - (_GRID - 1) * NVBLK


def _mm_body(wt_ref, p_ref, o_ref):
    o_ref[...] = lax.dot_general(
        wt_ref[...],
        p_ref[...],
        (((0,), (1,)), ((), ())),
        preferred_element_type=jnp.float32,
    )


_mm = pl.pallas_call(
    _mm_body,
    grid=(_GRID,),
    in_specs=[
        pl.BlockSpec((D, NVBLK), lambda i: (0, i)),
        pl.BlockSpec((B, D), lambda i: (0, 0)),
    ],
    out_specs=pl.BlockSpec((NVBLK, B), lambda i: (i, 0)),
    out_shape=jax.ShapeDtypeStruct((VOC, B), jnp.float32),
    compiler_params=pltpu.CompilerParams(
        dimension_semantics=("arbitrary",),
    ),
)


def kernel(X, W, WT):
    xr = X.astype(jnp.int32).reshape(NW, NCHUNK, ICHUNK)
    pooled = _pool_sc(xr, W)
    return _mm(WT.T, pooled).T


# speedup vs baseline: 1.1172x; 1.1172x over previous
import functools

import jax
import jax.numpy as jnp
from jax import lax
from jax.experimental import pallas as pl
from jax.experimental.pallas import tpu as pltpu
from jax.experimental.pallas import tpu_sc as plsc

VOC = 100000
D = 64
B = 1024
CTX = 20

NC = 2
NS = 16
NW = NC * NS
L = 16
BPW = B // NW
IPW = BPW * CTX
ICHUNK = 128
NCHUNK = IPW // ICHUNK
DPAD = 128

_sc_mesh = plsc.VectorSubcoreMesh(core_axis_name="c", subcore_axis_name="s")


@functools.partial(
    pl.kernel,
    mesh=_sc_mesh,
    out_type=jax.ShapeDtypeStruct((B, D), jnp.float32),
    scratch_types=[
        pltpu.VMEM((NCHUNK, ICHUNK), jnp.int32),
        pltpu.VMEM((IPW, DPAD), jnp.float32),
        pltpu.VMEM((BPW, D), jnp.float32),
        pltpu.SemaphoreType.DMA,
    ],
    compiler_params=pltpu.CompilerParams(use_tc_tiling_on_sc=False),
)
def _pool_sc(x_hbm, w_hbm, out_hbm, idx_v, rows_v, pool_v, sem):
    wid = lax.axis_index("s") * NC + lax.axis_index("c")
    pltpu.sync_copy(x_hbm.at[wid], idx_v)
    cps = [
        pltpu.async_copy(
            w_hbm.at[idx_v.at[j]],
            rows_v.at[pl.ds(j * ICHUNK, ICHUNK)],
            sem,
        )
        for j in range(NCHUNK)
    ]
    for cp in cps:
        cp.wait()

    inv_ctx = jnp.float32(1.0 / CTX)

    def body(b, carry):
        base = b * CTX
        for d in range(D // L):
            sl = pl.ds(d * L, L)
            acc = rows_v[base, sl]
            for j in range(1, CTX):
                acc = acc + rows_v[base + j, sl]
            pool_v[b, sl] = acc * inv_ctx
        return carry

    lax.fori_loop(0, BPW, body, 0)
    pltpu.sync_copy(pool_v, out_hbm.at[pl.ds(wid * BPW, BPW)])


NVBLK = 4096
_GRID = (VOC + NVBLK - 1) // NVBLK
_VOCPAD = ((VOC + 127) // 128) * 128
_LASTW = VOC - (_GRID - 1) * NVBLK


def _mm_body(wt_ref, p_ref, o_ref):
    o_ref[...] = lax.dot_general(
        wt_ref[...],
        p_ref[...],
        (((0,), (1,)), ((), ())),
        preferred_element_type=jnp.float32,
    )


_mm = pl.pallas_call(
    _mm_body,
    grid=(_GRID,),
    in_specs=[
        pl.BlockSpec((D, NVBLK), lambda i: (0, i)),
        pl.BlockSpec((B, D), lambda i: (0, 0)),
    ],
    out_specs=pl.BlockSpec((NVBLK, B), lambda i: (i, 0)),
    out_shape=jax.ShapeDtypeStruct((VOC, B), jnp.float32),
    compiler_params=pltpu.CompilerParams(
        dimension_semantics=("arbitrary",),
    ),
)


def _wprep_body(wt_ref, o_ref):
    o_ref[:, 0:D] = wt_ref[...].T


_wprep = pl.pallas_call(
    _wprep_body,
    grid=(_GRID,),
    in_specs=[pl.BlockSpec((D, NVBLK), lambda i: (0, i))],
    out_specs=pl.BlockSpec((NVBLK, DPAD), lambda i: (i, 0)),
    out_shape=jax.ShapeDtypeStruct((VOC, DPAD), jnp.float32),
    compiler_params=pltpu.CompilerParams(
        dimension_semantics=("arbitrary",),
    ),
)


def kernel(X, W, WT):
    xr = X.astype(jnp.int32).reshape(NW, NCHUNK, ICHUNK)
    wp = _wprep(W.T)
    pooled = _pool_sc(xr, wp)
    return _mm(WT.T, pooled).T


# speedup vs baseline: 1.1486x; 1.0281x over previous
import functools

import jax
import jax.numpy as jnp
from jax import lax
from jax.experimental import pallas as pl
from jax.experimental.pallas import tpu as pltpu
from jax.experimental.pallas import tpu_sc as plsc

VOC = 100000
D = 64
B = 1024
CTX = 20

NC = 2
NS = 16
NW = NC * NS
L = 16
BPW = B // NW
IPW = BPW * CTX
ICHUNK = 128
NCHUNK = IPW // ICHUNK
DPAD = 128

_sc_mesh = plsc.VectorSubcoreMesh(core_axis_name="c", subcore_axis_name="s")


@functools.partial(
    pl.kernel,
    mesh=_sc_mesh,
    out_type=jax.ShapeDtypeStruct((B, D), jnp.float32),
    scratch_types=[
        pltpu.VMEM((NCHUNK, ICHUNK), jnp.int32),
        pltpu.VMEM((IPW, DPAD), jnp.float32),
        pltpu.VMEM((BPW, D), jnp.float32),
        pltpu.SemaphoreType.DMA,
    ],
    compiler_params=pltpu.CompilerParams(use_tc_tiling_on_sc=False),
)
def _pool_sc(x_hbm, w_hbm, out_hbm, idx_v, rows_v, pool_v, sem):
    wid = lax.axis_index("s") * NC + lax.axis_index("c")
    pltpu.sync_copy(x_hbm.at[wid], idx_v)
    cps = [
        pltpu.async_copy(
            w_hbm.at[idx_v.at[j]],
            rows_v.at[pl.ds(j * ICHUNK, ICHUNK)],
            sem,
        )
        for j in range(NCHUNK)
    ]
    for cp in cps:
        cp.wait()

    inv_ctx = jnp.float32(1.0 / CTX)

    def body(b, carry):
        base = b * CTX
        for d in range(D // L):
            sl = pl.ds(d * L, L)
            acc = rows_v[base, sl]
            for j in range(1, CTX):
                acc = acc + rows_v[base + j, sl]
            pool_v[b, sl] = acc * inv_ctx
        return carry

    lax.fori_loop(0, BPW, body, 0)
    pltpu.sync_copy(pool_v, out_hbm.at[pl.ds(wid * BPW, BPW)])


NVBLK = 6144
_GRID = (VOC + NVBLK - 1) // NVBLK
_VOCPAD = ((VOC + 127) // 128) * 128
_LASTW = VOC - (_GRID - 1) * NVBLK


def _mm_body(wt_ref, p_ref, o_ref):
    o_ref[...] = lax.dot_general(
        wt_ref[...],
        p_ref[...],
        (((0,), (1,)), ((), ())),
        preferred_element_type=jnp.float32,
    )


_mm = pl.pallas_call(
    _mm_body,
    grid=(_GRID,),
    in_specs=[
        pl.BlockSpec((D, NVBLK), lambda i: (0, i)),
        pl.BlockSpec((B, D), lambda i: (0, 0)),
    ],
    out_specs=pl.BlockSpec((NVBLK, B), lambda i: (i, 0)),
    out_shape=jax.ShapeDtypeStruct((VOC, B), jnp.float32),
    compiler_params=pltpu.CompilerParams(
        dimension_semantics=("arbitrary",),
    ),
)


def _wprep_body(wt_ref, o_ref):
    o_ref[:, 0:D] = wt_ref[...].T


_wprep = pl.pallas_call(
    _wprep_body,
    grid=(_GRID,),
    in_specs=[pl.BlockSpec((D, NVBLK), lambda i: (0, i))],
    out_specs=pl.BlockSpec((NVBLK, DPAD), lambda i: (i, 0)),
    out_shape=jax.ShapeDtypeStruct((VOC, DPAD), jnp.float32),
    compiler_params=pltpu.CompilerParams(
        dimension_semantics=("arbitrary",),
    ),
)


def kernel(X, W, WT):
    xr = X.astype(jnp.int32).reshape(NW, NCHUNK, ICHUNK)
    wp = _wprep(W.T)
    pooled = _pool_sc(xr, wp)
    return _mm(WT.T, pooled).T
